# baseline (speedup 1.0000x reference)
import functools

import jax
import jax.numpy as jnp
from jax import lax
from jax.experimental import pallas as pl
from jax.experimental.pallas import tpu as pltpu
from jax.experimental.pallas import tpu_sc as plsc

B = 4096
D = 128
NUM_TABLES = 8
NIDX = NUM_TABLES * B
NC = 2
NS = 16
NW = NC * NS
B_PER_W = NIDX // NW
CHUNK = 256
BB = 512
HEX_BITS = 16
NUM_VOLATILE_FIELDS = 9


def _sc_gather(table, idx):
    mesh = plsc.VectorSubcoreMesh(core_axis_name="c", subcore_axis_name="s")

    @functools.partial(
        pl.kernel,
        out_type=jax.ShapeDtypeStruct((NIDX, D), jnp.float32),
        mesh=mesh,
        scratch_types=[
            pltpu.VMEM((CHUNK,), jnp.int32),
            pltpu.VMEM((CHUNK, D), jnp.float32),
            pltpu.SemaphoreType.DMA,
        ],
    )
    def gather_kernel(table_hbm, idx_hbm, out_hbm, idx_v, rows_v, sem):
        wid = lax.axis_index("s") * NC + lax.axis_index("c")
        base = wid * B_PER_W

        @pl.loop(0, B_PER_W, step=CHUNK)
        def _(off):
            pltpu.sync_copy(idx_hbm.at[pl.ds(base + off, CHUNK)], idx_v)
            pltpu.async_copy(table_hbm.at[idx_v], rows_v, sem).wait()
            pltpu.sync_copy(rows_v, out_hbm.at[pl.ds(base + off, CHUNK)])

    return gather_kernel(table, idx)


def _combine_body(rows_ref, vol_ref, sp_ref, whex_ref, wout_ref, b_ref, o_ref):
    g = rows_ref[...]
    acc = jnp.maximum(g[0], 0.0) + jnp.maximum(g[1], 0.0)
    acc += jnp.maximum(g[2], 0.0) + jnp.maximum(g[3], 0.0)
    acc += jnp.maximum((g[4] + g[5] + g[6] + g[7]) * 0.25, 0.0)
    v = vol_ref[...]
    k16 = lax.broadcasted_iota(jnp.int32, (1, HEX_BITS), 1)
    bits = jnp.concatenate(
        [jnp.right_shift(v[:, f : f + 1], k16) & 1
         for f in range(NUM_VOLATILE_FIELDS)],
        axis=1,
    ).astype(jnp.float32)
    acc += jnp.dot(bits, whex_ref[...], preferred_element_type=jnp.float32)
    out = jnp.dot(acc, wout_ref[...], preferred_element_type=jnp.float32)
    out = jnp.maximum(out + b_ref[...], 0.0)
    o_ref[...] = jnp.where(sp_ref[...] != 0, out, 0.0)


def _tc_combine(rows3, vol, sp, w_hex, w_out, b2):
    n_blocks = B // BB
    return pl.pallas_call(
        _combine_body,
        grid=(n_blocks,),
        in_specs=[
            pl.BlockSpec((NUM_TABLES, BB, D), lambda i: (0, i, 0)),
            pl.BlockSpec((BB, NUM_VOLATILE_FIELDS), lambda i: (i, 0)),
            pl.BlockSpec((BB, 1), lambda i: (i, 0)),
            pl.BlockSpec((NUM_VOLATILE_FIELDS * HEX_BITS, D), lambda i: (0, 0)),
            pl.BlockSpec((D, D), lambda i: (0, 0)),
            pl.BlockSpec((1, D), lambda i: (0, 0)),
        ],
        out_specs=pl.BlockSpec((BB, D), lambda i: (i, 0)),
        out_shape=jax.ShapeDtypeStruct((B, D), jnp.float32),
    )(rows3, vol, sp, w_hex, w_out, b2)


def kernel(species_idx, ability_idx, item_idx, side_idx, move_ids, volatiles,
           species_table, abilities_table, items_table, actions_table,
           side_table, W_hex, W_out, b_out):
    sp = species_idx.astype(jnp.int32)
    table = jnp.concatenate(
        [species_table, abilities_table, items_table, side_table,
         actions_table], axis=0)
    n_sp = species_table.shape[0]
    n_ab = abilities_table.shape[0]
    n_it = items_table.shape[0]
    n_sd = side_table.shape[0]
    off_ab = n_sp
    off_it = off_ab + n_ab
    off_sd = off_it + n_it
    off_ac = off_sd + n_sd
    idx = jnp.concatenate([
        sp,
        ability_idx.astype(jnp.int32) + off_ab,
        item_idx.astype(jnp.int32) + off_it,
        side_idx.astype(jnp.int32) + off_sd,
        (move_ids.astype(jnp.int32).T + off_ac).reshape(-1),
    ])
    rows = _sc_gather(table, idx)
    rows3 = rows.reshape(NUM_TABLES, B, D)
    return _tc_combine(rows3, volatiles.astype(jnp.int32),
                       sp.reshape(B, 1), W_hex, W_out,
                       b_out.reshape(1, D))


# speedup vs baseline: 1.0363x; 1.0363x over previous
import functools

import jax
import jax.numpy as jnp
from jax import lax
from jax.experimental import pallas as pl
from jax.experimental.pallas import tpu as pltpu
from jax.experimental.pallas import tpu_sc as plsc

B = 4096
D = 128
NUM_TABLES = 8
NIDX = NUM_TABLES * B
NC = 2
NS = 16
NW = NC * NS
B_PER_W = NIDX // NW
CHUNK = 256
BB = 512
HEX_BITS = 16
NUM_VOLATILE_FIELDS = 9


N_CHUNKS = B_PER_W // CHUNK


def _sc_gather(table, idx):
    mesh = plsc.VectorSubcoreMesh(core_axis_name="c", subcore_axis_name="s")

    @functools.partial(
        pl.kernel,
        out_type=jax.ShapeDtypeStruct((NIDX, D), jnp.float32),
        mesh=mesh,
        scratch_types=(
            [pltpu.VMEM((CHUNK,), jnp.int32) for _ in range(N_CHUNKS)]
            + [pltpu.VMEM((CHUNK, D), jnp.float32) for _ in range(2)]
            + [pltpu.SemaphoreType.DMA for _ in range(N_CHUNKS + 4)]
        ),
    )
    def gather_kernel(table_hbm, idx_hbm, out_hbm, *scratch):
        ib = scratch[:N_CHUNKS]
        rb = scratch[N_CHUNKS:N_CHUNKS + 2]
        sis = scratch[N_CHUNKS + 2:2 * N_CHUNKS + 2]
        sgs = scratch[2 * N_CHUNKS + 2:2 * N_CHUNKS + 4]
        sws = scratch[2 * N_CHUNKS + 4:2 * N_CHUNKS + 6]
        wid = lax.axis_index("s") * NC + lax.axis_index("c")
        base = wid * B_PER_W

        icp = [
            pltpu.async_copy(
                idx_hbm.at[pl.ds(base + k * CHUNK, CHUNK)], ib[k], sis[k])
            for k in range(N_CHUNKS)
        ]
        gcp = [None] * N_CHUNKS
        wcp = [None] * N_CHUNKS
        for k in range(N_CHUNKS):
            p = k % 2
            if k >= 2:
                wcp[k - 2].wait()
            icp[k].wait()
            gcp[k] = pltpu.async_copy(table_hbm.at[ib[k]], rb[p], sgs[p])
            if k >= 1:
                gcp[k - 1].wait()
                wcp[k - 1] = pltpu.async_copy(
                    rb[(k - 1) % 2],
                    out_hbm.at[pl.ds(base + (k - 1) * CHUNK, CHUNK)],
                    sws[(k - 1) % 2])
        gcp[N_CHUNKS - 1].wait()
        wcp[N_CHUNKS - 1] = pltpu.async_copy(
            rb[(N_CHUNKS - 1) % 2],
            out_hbm.at[pl.ds(base + (N_CHUNKS - 1) * CHUNK, CHUNK)],
            sws[(N_CHUNKS - 1) % 2])
        wcp[N_CHUNKS - 2].wait()
        wcp[N_CHUNKS - 1].wait()

    return gather_kernel(table, idx)


def _combine_body(rows_ref, vol_ref, sp_ref, whex_ref, wout_ref, b_ref, o_ref):
    g = rows_ref[...]
    acc = jnp.maximum(g[0], 0.0) + jnp.maximum(g[1], 0.0)
    acc += jnp.maximum(g[2], 0.0) + jnp.maximum(g[3], 0.0)
    acc += jnp.maximum((g[4] + g[5] + g[6] + g[7]) * 0.25, 0.0)
    v = vol_ref[...]
    k16 = lax.broadcasted_iota(jnp.int32, (1, HEX_BITS), 1)
    bits = jnp.concatenate(
        [jnp.right_shift(v[:, f : f + 1], k16) & 1
         for f in range(NUM_VOLATILE_FIELDS)],
        axis=1,
    ).astype(jnp.float32)
    acc += jnp.dot(bits, whex_ref[...], preferred_element_type=jnp.float32)
    out = jnp.dot(acc, wout_ref[...], preferred_element_type=jnp.float32)
    out = jnp.maximum(out + b_ref[...], 0.0)
    o_ref[...] = jnp.where(sp_ref[...] != 0, out, 0.0)


def _tc_combine(rows3, vol, sp, w_hex, w_out, b2):
    n_blocks = B // BB
    return pl.pallas_call(
        _combine_body,
        grid=(n_blocks,),
        in_specs=[
            pl.BlockSpec((NUM_TABLES, BB, D), lambda i: (0, i, 0)),
            pl.BlockSpec((BB, NUM_VOLATILE_FIELDS), lambda i: (i, 0)),
            pl.BlockSpec((BB, 1), lambda i: (i, 0)),
            pl.BlockSpec((NUM_VOLATILE_FIELDS * HEX_BITS, D), lambda i: (0, 0)),
            pl.BlockSpec((D, D), lambda i: (0, 0)),
            pl.BlockSpec((1, D), lambda i: (0, 0)),
        ],
        out_specs=pl.BlockSpec((BB, D), lambda i: (i, 0)),
        out_shape=jax.ShapeDtypeStruct((B, D), jnp.float32),
    )(rows3, vol, sp, w_hex, w_out, b2)


def kernel(species_idx, ability_idx, item_idx, side_idx, move_ids, volatiles,
           species_table, abilities_table, items_table, actions_table,
           side_table, W_hex, W_out, b_out):
    sp = species_idx.astype(jnp.int32)
    table = jnp.concatenate(
        [species_table, abilities_table, items_table, side_table,
         actions_table], axis=0)
    n_sp = species_table.shape[0]
    n_ab = abilities_table.shape[0]
    n_it = items_table.shape[0]
    n_sd = side_table.shape[0]
    off_ab = n_sp
    off_it = off_ab + n_ab
    off_sd = off_it + n_it
    off_ac = off_sd + n_sd
    idx = jnp.concatenate([
        sp,
        ability_idx.astype(jnp.int32) + off_ab,
        item_idx.astype(jnp.int32) + off_it,
        side_idx.astype(jnp.int32) + off_sd,
        (move_ids.astype(jnp.int32).T + off_ac).reshape(-1),
    ])
    rows = _sc_gather(table, idx)
    rows3 = rows.reshape(NUM_TABLES, B, D)
    return _tc_combine(rows3, volatiles.astype(jnp.int32),
                       sp.reshape(B, 1), W_hex, W_out,
                       b_out.reshape(1, D))
